# baseline (device time: 13858 ns/iter reference)
import jax
import jax.numpy as jnp
from jax import lax
from jax.experimental import pallas as pl
from jax.experimental.pallas import tpu as pltpu

C = 4


def kernel(partial, resid, gamma):
    _, m, d = partial.shape
    rows = m // C
    gamma2 = gamma.reshape(1, d)

    def body(p_hbm, r_hbm, g_ref, o_hbm, p_vmem, r_vmem, o_vmem,
             send_ref, comm_ref, load_sems, send_sems, recv_sems,
             store_sems):
        my_x = lax.axis_index("x")
        my_y = lax.axis_index("y")
        my_z = lax.axis_index("z")
        nbr = (1 - my_x, my_y, my_z)

        p_loads, r_loads = [], []
        for c in range(C):
            sl = slice(c * rows, (c + 1) * rows)
            cp = pltpu.make_async_copy(
                p_hbm.at[0, sl], p_vmem.at[sl], load_sems.at[0, c])
            cp.start()
            p_loads.append(cp)
            cr = pltpu.make_async_copy(
                r_hbm.at[sl], r_vmem.at[sl], load_sems.at[1, c])
            cr.start()
            r_loads.append(cr)

        barrier_sem = pltpu.get_barrier_semaphore()
        pl.semaphore_signal(
            barrier_sem, inc=1, device_id=nbr,
            device_id_type=pl.DeviceIdType.MESH,
        )
        pl.semaphore_wait(barrier_sem, 1)

        rdmas = []
        for c in range(C):
            sl = slice(c * rows, (c + 1) * rows)
            p_loads[c].wait()
            send_ref[sl, :] = p_vmem[sl, :].astype(jnp.bfloat16)
            rdma = pltpu.make_async_remote_copy(
                src_ref=send_ref.at[sl],
                dst_ref=comm_ref.at[sl],
                send_sem=send_sems.at[c],
                recv_sem=recv_sems.at[c],
                device_id=nbr,
                device_id_type=pl.DeviceIdType.MESH,
            )
            rdma.start()
            rdmas.append(rdma)

        stores = []
        for c in range(C):
            rdmas[c].wait_recv()
            r_loads[c].wait()
            sl = slice(c * rows, (c + 1) * rows)
            y = (p_vmem[sl, :]
                 + comm_ref[sl, :].astype(jnp.float32)
                 + r_vmem[sl, :])
            rms = jnp.sqrt(jnp.mean(y * y, axis=-1, keepdims=True) + 1e-6)
            o_vmem[sl, :] = y / rms * g_ref[...]
            st = pltpu.make_async_copy(
                o_vmem.at[sl], o_hbm.at[sl], store_sems.at[c])
            st.start()
            stores.append(st)

        for c in range(C):
            stores[c].wait()
            rdmas[c].wait_send()

    return pl.pallas_call(
        body,
        out_shape=jax.ShapeDtypeStruct((m, d), jnp.float32),
        in_specs=[
            pl.BlockSpec(memory_space=pl.ANY),
            pl.BlockSpec(memory_space=pl.ANY),
            pl.BlockSpec(memory_space=pltpu.VMEM),
        ],
        out_specs=pl.BlockSpec(memory_space=pl.ANY),
        scratch_shapes=[
            pltpu.VMEM((m, d), jnp.float32),
            pltpu.VMEM((m, d), jnp.float32),
            pltpu.VMEM((m, d), jnp.float32),
            pltpu.VMEM((m, d), jnp.bfloat16),
            pltpu.VMEM((m, d), jnp.bfloat16),
            pltpu.SemaphoreType.DMA((2, C)),
            pltpu.SemaphoreType.DMA((C,)),
            pltpu.SemaphoreType.DMA((C,)),
            pltpu.SemaphoreType.DMA((C,)),
        ],
        compiler_params=pltpu.CompilerParams(collective_id=0),
    )(partial, resid, gamma2)


# device time: 13673 ns/iter; 1.0135x vs baseline; 1.0135x over previous
import jax
import jax.numpy as jnp
from jax import lax
from jax.experimental import pallas as pl
from jax.experimental.pallas import tpu as pltpu

C = 8


def kernel(partial, resid, gamma):
    _, m, d = partial.shape
    rows = m // C
    gamma2 = gamma.reshape(1, d)

    def body(p_ref, r_ref, g_ref, o_ref, send_ref, comm_ref,
             send_sems, recv_sems):
        my_x = lax.axis_index("x")
        my_y = lax.axis_index("y")
        my_z = lax.axis_index("z")
        nbr = (1 - my_x, my_y, my_z)

        send_ref[...] = p_ref[0].astype(jnp.bfloat16)

        barrier_sem = pltpu.get_barrier_semaphore()
        pl.semaphore_signal(
            barrier_sem, inc=1, device_id=nbr,
            device_id_type=pl.DeviceIdType.MESH,
        )
        pl.semaphore_wait(barrier_sem, 1)

        rdmas = []
        for c in range(C):
            sl = slice(c * rows, (c + 1) * rows)
            rdma = pltpu.make_async_remote_copy(
                src_ref=send_ref.at[sl],
                dst_ref=comm_ref.at[sl],
                send_sem=send_sems.at[c],
                recv_sem=recv_sems.at[c],
                device_id=nbr,
                device_id_type=pl.DeviceIdType.MESH,
            )
            rdma.start()
            rdmas.append(rdma)

        for c in range(C):
            rdmas[c].wait_recv()
            sl = slice(c * rows, (c + 1) * rows)
            y = (p_ref[0, sl, :]
                 + comm_ref[sl, :].astype(jnp.float32)
                 + r_ref[sl, :])
            rms = jnp.sqrt(jnp.mean(y * y, axis=-1, keepdims=True) + 1e-6)
            o_ref[sl, :] = y / rms * g_ref[...]

        for c in range(C):
            rdmas[c].wait_send()

    return pl.pallas_call(
        body,
        out_shape=jax.ShapeDtypeStruct((m, d), jnp.float32),
        in_specs=[
            pl.BlockSpec(memory_space=pltpu.VMEM),
            pl.BlockSpec(memory_space=pltpu.VMEM),
            pl.BlockSpec(memory_space=pltpu.VMEM),
        ],
        out_specs=pl.BlockSpec(memory_space=pltpu.VMEM),
        scratch_shapes=[
            pltpu.VMEM((m, d), jnp.bfloat16),
            pltpu.VMEM((m, d), jnp.bfloat16),
            pltpu.SemaphoreType.DMA((C,)),
            pltpu.SemaphoreType.DMA((C,)),
        ],
        compiler_params=pltpu.CompilerParams(collective_id=0),
    )(partial, resid, gamma2)
